# baseline (speedup 1.0000x reference)
import functools

import jax
import jax.numpy as jnp
from jax import lax
from jax.experimental import pallas as pl
from jax.experimental.pallas import tpu as pltpu
from jax.experimental.pallas import tpu_sc as plsc

_NUM_CORES = 2
_NUM_SUBCORES = 16
_NUM_WORKERS = _NUM_CORES * _NUM_SUBCORES
_CHUNKS = 8


@jax.jit
def _embed(x, W):
    B, = x.shape
    V, D = W.shape
    b_per_w = B // _NUM_WORKERS
    cpw = b_per_w // _CHUNKS
    ngrp = D // 16

    mesh = plsc.VectorSubcoreMesh(core_axis_name="c", subcore_axis_name="s")

    @functools.partial(
        pl.kernel,
        mesh=mesh,
        out_type=jax.ShapeDtypeStruct((B, D), jnp.float32),
        scratch_types=[
            pltpu.VMEM((b_per_w,), jnp.int32),
            pltpu.VMEM((V, D), jnp.float32),
            pltpu.VMEM((2, cpw, D), jnp.float32),
            pltpu.SemaphoreType.DMA,
            pltpu.SemaphoreType.DMA,
        ],
    )
    def k(x_hbm, w_hbm, out_hbm, idx_v, table_v, out_v, wsem0, wsem1):
        wid = lax.axis_index("s") * _NUM_CORES + lax.axis_index("c")
        base = wid * b_per_w
        wsems = (wsem0, wsem1)

        pltpu.sync_copy(w_hbm, table_v)
        pltpu.sync_copy(x_hbm.at[pl.ds(base, b_per_w)], idx_v)

        wh = [None] * _CHUNKS
        for c in range(_CHUNKS):
            buf = c % 2
            if c >= 2:
                wh[c - 2].wait()

            def body(rb, carry, c=c, buf=buf):
                idxv = idx_v[pl.ds(c * cpw + rb * 16, 16)]
                rbase = rb * 16
                rows = [idxv[l] for l in range(16)]
                depth = 8
                pend = []
                for l in range(16):
                    for g in range(ngrp):
                        val = table_v[rows[l], pl.ds(g * 16, 16)]
                        pend.append((l, g, val))
                        if len(pend) > depth:
                            l2, g2, v2 = pend.pop(0)
                            out_v[buf, rbase + l2, pl.ds(g2 * 16, 16)] = v2
                for l2, g2, v2 in pend:
                    out_v[buf, rbase + l2, pl.ds(g2 * 16, 16)] = v2
                return carry

            lax.fori_loop(0, cpw // 16, body, None, unroll=2)
            wh[c] = pltpu.async_copy(
                out_v.at[buf], out_hbm.at[pl.ds(base + c * cpw, cpw)],
                wsems[buf])
        wh[_CHUNKS - 2].wait()
        wh[_CHUNKS - 1].wait()

    return k(x, W)


def kernel(x, W):
    return _embed(x.astype(jnp.int32), W)


# speedup vs baseline: 1.1182x; 1.1182x over previous
import functools

import jax
import jax.numpy as jnp
from jax import lax
from jax.experimental import pallas as pl
from jax.experimental.pallas import tpu as pltpu
from jax.experimental.pallas import tpu_sc as plsc

_NUM_CORES = 2
_NUM_SUBCORES = 16
_NUM_WORKERS = _NUM_CORES * _NUM_SUBCORES
_CHUNKS = 2


@jax.jit
def _embed(x, W):
    B, = x.shape
    V, D = W.shape
    b_per_w = B // _NUM_WORKERS
    cpw = b_per_w // _CHUNKS
    ngrp = D // 16

    mesh = plsc.VectorSubcoreMesh(core_axis_name="c", subcore_axis_name="s")

    @functools.partial(
        pl.kernel,
        mesh=mesh,
        out_type=jax.ShapeDtypeStruct((B, D), jnp.float32),
        scratch_types=[
            pltpu.VMEM((b_per_w,), jnp.int32),
            pltpu.VMEM((V, D), jnp.float32),
            pltpu.VMEM((2, cpw, D), jnp.float32),
            pltpu.SemaphoreType.DMA,
            pltpu.SemaphoreType.DMA,
        ],
    )
    def k(x_hbm, w_hbm, out_hbm, idx_v, table_v, out_v, wsem0, wsem1):
        wid = lax.axis_index("s") * _NUM_CORES + lax.axis_index("c")
        base = wid * b_per_w
        wsems = (wsem0, wsem1)

        pltpu.sync_copy(w_hbm, table_v)
        pltpu.sync_copy(x_hbm.at[pl.ds(base, b_per_w)], idx_v)

        wh = [None] * _CHUNKS
        for c in range(_CHUNKS):
            buf = c % 2
            if c >= 2:
                wh[c - 2].wait()

            def body(rb, carry, c=c, buf=buf):
                idxv = idx_v[pl.ds(c * cpw + rb * 16, 16)]
                rbase = rb * 16
                rows = [idxv[l] for l in range(16)]
                depth = 8
                pend = []
                for l in range(16):
                    for g in range(ngrp):
                        val = table_v[rows[l], pl.ds(g * 16, 16)]
                        pend.append((l, g, val))
                        if len(pend) > depth:
                            l2, g2, v2 = pend.pop(0)
                            out_v[buf, rbase + l2, pl.ds(g2 * 16, 16)] = v2
                for l2, g2, v2 in pend:
                    out_v[buf, rbase + l2, pl.ds(g2 * 16, 16)] = v2
                return carry

            lax.fori_loop(0, cpw // 16, body, None, unroll=2)
            wh[c] = pltpu.async_copy(
                out_v.at[buf], out_hbm.at[pl.ds(base + c * cpw, cpw)],
                wsems[buf])
        wh[_CHUNKS - 2].wait()
        wh[_CHUNKS - 1].wait()

    return k(x, W)


def kernel(x, W):
    return _embed(x.astype(jnp.int32), W)


# speedup vs baseline: 1.1390x; 1.0187x over previous
import functools

import jax
import jax.numpy as jnp
from jax import lax
from jax.experimental import pallas as pl
from jax.experimental.pallas import tpu as pltpu
from jax.experimental.pallas import tpu_sc as plsc

_NUM_CORES = 2
_NUM_SUBCORES = 16
_NUM_WORKERS = _NUM_CORES * _NUM_SUBCORES
_CHUNKS = 2


@jax.jit
def _embed(x, W):
    B, = x.shape
    V, D = W.shape
    b_per_w = B // _NUM_WORKERS
    cpw = b_per_w // _CHUNKS
    ngrp = D // 16

    mesh = plsc.VectorSubcoreMesh(core_axis_name="c", subcore_axis_name="s")

    @functools.partial(
        pl.kernel,
        mesh=mesh,
        out_type=jax.ShapeDtypeStruct((B, D), jnp.float32),
        scratch_types=[
            pltpu.VMEM((b_per_w,), jnp.int32),
            pltpu.VMEM((V, D), jnp.float32),
            pltpu.VMEM((2, cpw, D), jnp.float32),
            pltpu.SemaphoreType.DMA,
            pltpu.SemaphoreType.DMA,
        ],
    )
    def k(x_hbm, w_hbm, out_hbm, idx_v, table_v, out_v, wsem0, wsem1):
        wid = lax.axis_index("s") * _NUM_CORES + lax.axis_index("c")
        base = wid * b_per_w
        wsems = (wsem0, wsem1)

        th = pltpu.async_copy(w_hbm, table_v, wsem0)
        ih = pltpu.async_copy(x_hbm.at[pl.ds(base, b_per_w)], idx_v, wsem1)
        th.wait()
        ih.wait()

        wh = [None] * _CHUNKS
        for c in range(_CHUNKS):
            buf = c % 2
            if c >= 2:
                wh[c - 2].wait()

            def body(rb, carry, c=c, buf=buf):
                idxv = idx_v[pl.ds(c * cpw + rb * 16, 16)]
                rbase = rb * 16
                rows = [idxv[l] for l in range(16)]
                depth = 8
                pend = []
                for l in range(16):
                    for g in range(ngrp):
                        val = table_v[rows[l], pl.ds(g * 16, 16)]
                        pend.append((l, g, val))
                        if len(pend) > depth:
                            l2, g2, v2 = pend.pop(0)
                            out_v[buf, rbase + l2, pl.ds(g2 * 16, 16)] = v2
                for l2, g2, v2 in pend:
                    out_v[buf, rbase + l2, pl.ds(g2 * 16, 16)] = v2
                return carry

            lax.fori_loop(0, cpw // 16, body, None, unroll=2)
            wh[c] = pltpu.async_copy(
                out_v.at[buf], out_hbm.at[pl.ds(base + c * cpw, cpw)],
                wsems[buf])
        wh[_CHUNKS - 2].wait()
        wh[_CHUNKS - 1].wait()

    return k(x, W)


def kernel(x, W):
    return _embed(x.astype(jnp.int32), W)
